# baseline (speedup 1.0000x reference)
import functools

import jax
import jax.numpy as jnp
from jax import lax
from jax.experimental import pallas as pl
from jax.experimental.pallas import tpu as pltpu
from jax.experimental.pallas import tpu_sc as plsc

_N = 10000
_NPAD = 10240
_BN = 512
_NB = _NPAD // _BN
_E = 160000
_EPAD = 163840
_NCH = 80
_CH = 64
_TAU = 100.0
_F32 = jnp.float32
_PREC = jax.lax.Precision.HIGHEST


def _bdot(a, b):
    return jnp.dot(a.astype(jnp.bfloat16), b.astype(jnp.bfloat16),
                   preferred_element_type=_F32)


def _enc_body(xcyc, w0, b0, w1, b1, w2, b2, out):
    h = jnp.maximum(_bdot(xcyc[...], w0[...]) + b0[...], 0.0)
    h = jnp.maximum(_bdot(h, w1[...]) + b1[...], 0.0)
    out[...] = _bdot(h, w2[...]) + b2[...]


def _stats_body(xcT, pos_blk, m_out, z_out, m_sc, z_sc):
    i = pl.program_id(0)

    @pl.when(i == 0)
    def _():
        m_sc[...] = jnp.full_like(m_sc[...], -1e30)
        z_sc[...] = jnp.zeros_like(z_sc[...])

    px = pos_blk[:, 0:1]
    py = pos_blk[:, 1:2]
    xr = xcT[0:1, :]
    yr = xcT[1:2, :]
    xp = _bdot(pos_blk[...], xcT[0:2, :])
    d2 = (xr * xr + yr * yr + (px * px + py * py)) - 2.0 * xp
    logits = -_TAU * d2
    bm = jnp.max(logits, axis=0, keepdims=True)
    m_old = m_sc[...]
    m_new = jnp.maximum(m_old, bm)
    alpha = jnp.exp(m_old - m_new)
    z_sc[...] = z_sc[...] * alpha + jnp.sum(jnp.exp(logits - m_new), axis=0,
                                            keepdims=True)
    m_sc[...] = m_new

    @pl.when(i == _NB - 1)
    def _():
        m_out[...] = m_sc[...]
        z_out[...] = z_sc[...]


def _agg_body(xcT, pos_blk, emb, z_row, m_row, pose_blk, out_blk):
    px = pos_blk[:, 0:1]
    py = pos_blk[:, 1:2]
    xr = xcT[0:1, :]
    yr = xcT[1:2, :]
    xp = _bdot(pos_blk[...], xcT[0:2, :])
    d2 = (xr * xr + yr * yr + (px * px + py * py)) - 2.0 * xp
    w = jnp.exp(-_TAU * d2 - m_row[...]) / z_row[...]
    out_blk[...] = _bdot(w, emb[...]) + pose_blk[...]


def _pack_bf(x):
    xi = lax.bitcast_convert_type(x, jnp.int32) + jnp.int32(0x8000)
    xi = jnp.bitwise_and(xi, jnp.int32(-65536))
    lo = xi[:, :64]
    hi = lax.shift_right_logical(xi[:, 64:], 16)
    return jnp.bitwise_or(lo, hi)


def _as_body(nodes_blk, wrT, wsT, mb, a_out, s_out):
    nd = nodes_blk[...]
    a_out[...] = _pack_bf(_bdot(nd, wrT[...]) + mb[...])
    s_out[...] = _pack_bf(_bdot(nd, wsT[...]))


def _upd_body(nodes_blk, ib_blk, w1T, w2T, nbias, g2, b2, pose_blk, out_blk):
    nd = nodes_blk[...]
    ib = ib_blk[0] + ib_blk[1]
    y = _bdot(nd, w1T[...]) + _bdot(ib, w2T[...]) + nbias[...]
    mu = jnp.sum(y, axis=1, keepdims=True) * (1.0 / 126.0)
    var = jnp.sum(y * y, axis=1, keepdims=True) * (1.0 / 126.0) - mu * mu
    rstd = lax.rsqrt(var + 1e-5)
    out_blk[...] = (y - mu) * rstd * g2[...] + b2[...] + pose_blk[...]


def _tgt_body(xt, posT_blk, lat_blk, xte, m_col, z_col,
              dw0, db0, dw1, db1, dw2, db2, out, acc_sc):
    i = pl.program_id(0)

    @pl.when(i == 0)
    def _():
        acc_sc[...] = jnp.zeros_like(acc_sc[...])

    xa = xt[:, 0:1]
    ya = xt[:, 1:2]
    pxr = posT_blk[0:1, :]
    pyr = posT_blk[1:2, :]
    xp = _bdot(xt[...], posT_blk[0:2, :])
    d2 = (xa * xa + ya * ya + (pxr * pxr + pyr * pyr)) - 2.0 * xp
    p = jnp.exp(-_TAU * d2 - m_col[...]) / z_col[...]
    acc_sc[...] = acc_sc[...] + _bdot(p, lat_blk[...])

    @pl.when(i == _NB - 1)
    def _():
        colmask = (lax.broadcasted_iota(jnp.int32, (1, 128), 1) >= 2
                   ).astype(_F32)
        zin = acc_sc[...] * colmask + xte[...]
        h = jnp.maximum(_bdot(zin, dw0[...]) + db0[...], 0.0)
        h = jnp.maximum(_bdot(h, dw1[...]) + db1[...], 0.0)
        out[...] = _bdot(h, dw2[...]) + db2[...]


def _rsqrt16(v):
    i = plsc.bitcast(v, jnp.int32)
    i = jnp.full((16,), 0x5F3759DF, jnp.int32) - jnp.right_shift(
        i, jnp.full((16,), 1, jnp.int32))
    y = plsc.bitcast(i, _F32)
    halfv = v * jnp.full((16,), 0.5, _F32)
    c15 = jnp.full((16,), 1.5, _F32)
    for _ in range(3):
        y = y * (c15 - halfv * y * y)
    return y


def _sc_body(a_t, s_t, ridx_h, sidx_h, g_h, b_h, out_h,
             ridx_v, sidx_v, a0, a1, s0, s1, msg, gb_v,
             red0, red1, red2, red3, spm, ga0, ga1, gs0, gs1):
    core = lax.axis_index("c")
    sub = lax.axis_index("s")

    def _zrow(r, c):
        for k in range(8):
            msg[r, pl.ds(k * 16, 16)] = jnp.zeros((16,), _F32)
        return c
    lax.fori_loop(0, _CH, _zrow, 0)
    for t in range(640 // _CH):
        pltpu.sync_copy(msg, spm.at[pl.ds(sub * 640 + t * _CH, _CH)])
    plsc.subcore_barrier()

    pltpu.sync_copy(g_h, gb_v.at[0])
    pltpu.sync_copy(b_h, gb_v.at[1])
    gs = [gb_v[0, pl.ds(k * 16, 16)] for k in range(8)]
    bs = [gb_v[1, pl.ds(k * 16, 16)] for k in range(8)]

    pltpu.sync_copy(ridx_h.at[core, sub], ridx_v)
    pltpu.sync_copy(sidx_h.at[core, sub, pl.ds(0, _NCH // 2)], sidx_v)

    iota = lax.iota(jnp.int32, 16)
    perms = [jnp.bitwise_xor(iota, jnp.full((16,), k, jnp.int32))
             for k in (8, 4, 2, 1)]
    c_inv128 = jnp.full((16,), 1.0 / 128.0, _F32)
    c_eps = jnp.full((16,), 1e-5, _F32)

    def _allsum(v, slot):
        for p in perms:
            slot[...] = v
            v = v + plsc.load_gather(slot, [p])
        return v

    mask_hi = jnp.full((16,), -65536, jnp.int32)
    c16 = jnp.full((16,), 16, jnp.int32)

    def _ln_rows(av, sv, ra, rb):
        def _row(r, cc):
            x = [None] * 8
            for k in range(4):
                pa = av[r, pl.ds(k * 16, 16)]
                ps = sv[r, pl.ds(k * 16, 16)]
                x[k] = (plsc.bitcast(jnp.bitwise_and(pa, mask_hi), _F32)
                        + plsc.bitcast(jnp.bitwise_and(ps, mask_hi), _F32))
                x[4 + k] = (plsc.bitcast(lax.shift_left(pa, c16), _F32)
                            + plsc.bitcast(lax.shift_left(ps, c16), _F32))
            s = (((x[0] + x[1]) + (x[2] + x[3]))
                 + ((x[4] + x[5]) + (x[6] + x[7])))
            q = x[0] * x[0]
            for k in range(1, 8):
                q = q + x[k] * x[k]
            tot = _allsum(s, ra)
            qt = _allsum(q, rb)
            mu = tot * c_inv128
            var = qt * c_inv128 - mu * mu
            rv = _rsqrt16(var + c_eps)
            for k in range(8):
                msg[r, pl.ds(k * 16, 16)] = (x[k] - mu) * (gs[k] * rv) + bs[k]
            return cc
        lax.fori_loop(0, _CH, _row, 0)

    def _srow(jj):
        return jnp.where(jj >= _NCH // 2, jj - _NCH // 2, jj)

    pltpu.async_copy(a_t.at[ridx_v.at[0]], a0, ga0)
    pltpu.async_copy(s_t.at[sidx_v.at[0]], s0, gs0)

    def _pair(t, c):
        j0 = t * 2
        j1 = j0 + 1
        pltpu.make_async_copy(a_t.at[ridx_v.at[0]], a0, ga0).wait()
        pltpu.make_async_copy(s_t.at[sidx_v.at[0]], s0, gs0).wait()
        pltpu.async_copy(a_t.at[ridx_v.at[j1]], a1, ga1)
        pltpu.async_copy(s_t.at[sidx_v.at[_srow(j1)]], s1, gs1)
        _ln_rows(a0, s0, red0, red1)
        pltpu.sync_copy(msg, spm.at[ridx_v.at[j0]], add=True)

        pltpu.make_async_copy(a_t.at[ridx_v.at[0]], a1, ga1).wait()
        pltpu.make_async_copy(s_t.at[sidx_v.at[0]], s1, gs1).wait()

        @pl.when(t == _NCH // 4 - 1)
        def _():
            pltpu.sync_copy(sidx_h.at[core, sub, pl.ds(_NCH // 2, _NCH // 2)],
                            sidx_v)

        @pl.when(t < _NCH // 2 - 1)
        def _():
            j2 = j0 + 2
            pltpu.async_copy(a_t.at[ridx_v.at[j2]], a0, ga0)
            pltpu.async_copy(s_t.at[sidx_v.at[_srow(j2)]], s0, gs0)
        _ln_rows(a1, s1, red2, red3)
        pltpu.sync_copy(msg, spm.at[ridx_v.at[j1]], add=True)
        return c
    lax.fori_loop(0, _NCH // 2, _pair, 0)

    plsc.subcore_barrier()
    pltpu.sync_copy(spm.at[pl.ds(sub * 640, 640)],
                    out_h.at[core, pl.ds(sub * 640, 640)])


def _edge_phase(a_t, s_t, ridx3, sidx3, g, b):
    mesh = plsc.VectorSubcoreMesh(core_axis_name="c", subcore_axis_name="s",
                                  num_cores=2, num_subcores=16)
    f = pl.kernel(
        _sc_body,
        out_type=jax.ShapeDtypeStruct((2, _NPAD, 128), _F32),
        mesh=mesh,
        scratch_types=[
            pltpu.VMEM((_NCH, _CH), jnp.int32),
            pltpu.VMEM((_NCH // 2, _CH), jnp.int32),
            pltpu.VMEM((_CH, 64), jnp.int32),
            pltpu.VMEM((_CH, 64), jnp.int32),
            pltpu.VMEM((_CH, 64), jnp.int32),
            pltpu.VMEM((_CH, 64), jnp.int32),
            pltpu.VMEM((_CH, 128), _F32),
            pltpu.VMEM((2, 128), _F32),
            pltpu.VMEM((16,), _F32),
            pltpu.VMEM((16,), _F32),
            pltpu.VMEM((16,), _F32),
            pltpu.VMEM((16,), _F32),
            pltpu.VMEM_SHARED((_NPAD, 128), _F32),
            pltpu.SemaphoreType.DMA,
            pltpu.SemaphoreType.DMA,
            pltpu.SemaphoreType.DMA,
            pltpu.SemaphoreType.DMA,
        ],
        compiler_params=pltpu.CompilerParams(needs_layout_passes=False,
                                             use_tc_tiling_on_sc=False),
    )
    return f(a_t, s_t, ridx3, sidx3, g, b)


def _pad2d(a, rows, cols, row0=0, col0=0, fill=0.0):
    out = jnp.full((rows, cols), fill, a.dtype)
    return out.at[row0:row0 + a.shape[0], col0:col0 + a.shape[1]].set(a)


def kernel(xc, yc, xt, pos, senders, receivers,
           enc_W0, enc_b0, enc_W1, enc_b1, enc_W2, enc_b2,
           dec_W0, dec_b0, dec_W1, dec_b1, dec_W2, dec_b2,
           msg_W, msg_b, node_W, node_b,
           ln1_g, ln1_b, ln2_g, ln2_b):
    xc0, yc0, xt0 = xc[0], yc[0], xt[0]
    nc = xc0.shape[0]

    pos_far = jnp.concatenate(
        [pos, jnp.full((_NPAD - _N, 2), 1e3, _F32)], axis=0)
    pos_embed = _pad2d(pos, _NPAD, 128)
    xcT8 = _pad2d(xc0.T, 8, nc)
    posT8 = _pad2d(pos_far.T, 8, _NPAD)

    xtT8 = _pad2d(xt0.T, 8, nc)
    xcyc = _pad2d(jnp.concatenate([xc0, yc0], axis=1), nc, 8)
    ew0 = _pad2d(enc_W0.T, 8, 128)
    eb0 = _pad2d(enc_b0[None], 1, 128)
    ew1 = _pad2d(enc_W1.T, 128, 128)
    eb1 = _pad2d(enc_b1[None], 1, 128)
    ew2s = _pad2d(enc_W2.T, 128, 128, col0=2)
    eb2s = _pad2d(enc_b2[None], 1, 128, col0=2)

    mWT = msg_W.T
    wrT, wsT = mWT[:128], mWT[128:]
    mb = msg_b[None]

    nWT = node_W.T
    w1Ts = _pad2d(nWT[:128], 128, 128, col0=2)
    w2Ts = _pad2d(nWT[128:], 128, 128, col0=2)
    nbias = _pad2d(node_b[None], 1, 128, col0=2)
    g2s = _pad2d(ln2_g[None], 1, 128, col0=2)
    b2s = _pad2d(ln2_b[None], 1, 128, col0=2)

    dw0T = dec_W0.T
    dw0r = _pad2d(jnp.concatenate([dw0T[126:], dw0T[:126]], axis=0), 128, 128)
    db0 = _pad2d(dec_b0[None], 1, 128)
    dw1 = _pad2d(dec_W1.T, 128, 128)
    db1 = _pad2d(dec_b1[None], 1, 128)
    dw2 = _pad2d(dec_W2.T, 128, 128)
    db2 = _pad2d(dec_b2[None], 1, 128)
    xte = _pad2d(xt0, nc, 128)

    pad_e = _EPAD - _E
    send3 = jnp.concatenate(
        [senders, jnp.zeros((pad_e,), jnp.int32)]).reshape(2, 16, _NCH, _CH)
    recv3 = jnp.concatenate(
        [receivers, jnp.full((pad_e,), _N, jnp.int32)]).reshape(2, 16, _NCH, _CH)

    emb = pl.pallas_call(
        _enc_body,
        out_shape=jax.ShapeDtypeStruct((nc, 128), _F32),
    )(xcyc, ew0, eb0, ew1, eb1, ew2s, eb2s)

    full = lambda *_: (0, 0)

    def _stats(xT8):
        return pl.pallas_call(
            _stats_body,
            grid=(_NB,),
            in_specs=[pl.BlockSpec((8, nc), full),
                      pl.BlockSpec((_BN, 2), lambda i: (i, 0))],
            out_specs=[pl.BlockSpec((1, nc), full), pl.BlockSpec((1, nc), full)],
            out_shape=[jax.ShapeDtypeStruct((1, nc), _F32)] * 2,
            scratch_shapes=[pltpu.VMEM((1, nc), _F32),
                            pltpu.VMEM((1, nc), _F32)],
        )(xT8, pos_far)

    m_row, z_row = _stats(xcT8)
    m_rowt, z_rowt = _stats(xtT8)
    m_colt = m_rowt.reshape(nc, 1)
    z_colt = z_rowt.reshape(nc, 1)

    nodes = pl.pallas_call(
        _agg_body,
        grid=(_NB,),
        in_specs=[pl.BlockSpec((8, nc), full),
                  pl.BlockSpec((_BN, 2), lambda i: (i, 0)),
                  pl.BlockSpec((nc, 128), full),
                  pl.BlockSpec((1, nc), full),
                  pl.BlockSpec((1, nc), full),
                  pl.BlockSpec((_BN, 128), lambda i: (i, 0))],
        out_specs=pl.BlockSpec((_BN, 128), lambda i: (i, 0)),
        out_shape=jax.ShapeDtypeStruct((_NPAD, 128), _F32),
    )(xcT8, pos_far, emb, z_row, m_row, pos_embed)

    for _ in range(3):
        a_t, s_t = pl.pallas_call(
            _as_body,
            grid=(_NB,),
            in_specs=[pl.BlockSpec((_BN, 128), lambda i: (i, 0)),
                      pl.BlockSpec((128, 128), full),
                      pl.BlockSpec((128, 128), full),
                      pl.BlockSpec((1, 128), full)],
            out_specs=[pl.BlockSpec((_BN, 64), lambda i: (i, 0))] * 2,
            out_shape=[jax.ShapeDtypeStruct((_NPAD, 64), jnp.int32)] * 2,
        )(nodes, wrT, wsT, mb)

        inbox2 = _edge_phase(a_t, s_t, recv3, send3, ln1_g, ln1_b)

        nodes = pl.pallas_call(
            _upd_body,
            grid=(_NB,),
            in_specs=[pl.BlockSpec((_BN, 128), lambda i: (i, 0)),
                      pl.BlockSpec((2, _BN, 128), lambda i: (0, i, 0)),
                      pl.BlockSpec((128, 128), full),
                      pl.BlockSpec((128, 128), full),
                      pl.BlockSpec((1, 128), full),
                      pl.BlockSpec((1, 128), full),
                      pl.BlockSpec((1, 128), full),
                      pl.BlockSpec((_BN, 128), lambda i: (i, 0))],
            out_specs=pl.BlockSpec((_BN, 128), lambda i: (i, 0)),
            out_shape=jax.ShapeDtypeStruct((_NPAD, 128), _F32),
        )(nodes, inbox2, w1Ts, w2Ts, nbias, g2s, b2s, pos_embed)

    outp = pl.pallas_call(
        _tgt_body,
        grid=(_NB,),
        in_specs=[pl.BlockSpec((nc, 2), full),
                  pl.BlockSpec((8, _BN), lambda i: (0, i)),
                  pl.BlockSpec((_BN, 128), lambda i: (i, 0)),
                  pl.BlockSpec((nc, 128), full),
                  pl.BlockSpec((nc, 1), full),
                  pl.BlockSpec((nc, 1), full)]
                 + [pl.BlockSpec((128, 128), full),
                    pl.BlockSpec((1, 128), full)] * 3,
        out_specs=pl.BlockSpec((nc, 128), full),
        out_shape=jax.ShapeDtypeStruct((nc, 128), _F32),
        scratch_shapes=[pltpu.VMEM((nc, 128), _F32)],
    )(xt0, posT8, nodes, xte, m_colt, z_colt, dw0r, db0, dw1, db1, dw2, db2)

    return outp[:, :3][None]


# speedup vs baseline: 1.0880x; 1.0880x over previous
import functools

import jax
import jax.numpy as jnp
from jax import lax
from jax.experimental import pallas as pl
from jax.experimental.pallas import tpu as pltpu
from jax.experimental.pallas import tpu_sc as plsc

_N = 10000
_NPAD = 10240
_BN = 512
_NB = _NPAD // _BN
_E = 160000
_EPAD = 163840
_NCH = 80
_CH = 64
_TAU = 100.0
_F32 = jnp.float32
_PREC = jax.lax.Precision.HIGHEST


def _bdot(a, b):
    return jnp.dot(a.astype(jnp.bfloat16), b.astype(jnp.bfloat16),
                   preferred_element_type=_F32)


def _enc_body(xcyc, w0, b0, w1, b1, w2, b2, out):
    h = jnp.maximum(_bdot(xcyc[...], w0[...]) + b0[...], 0.0)
    h = jnp.maximum(_bdot(h, w1[...]) + b1[...], 0.0)
    out[...] = _bdot(h, w2[...]) + b2[...]


def _stats_body(xcT, pos_blk, m_out, z_out, m_sc, z_sc):
    i = pl.program_id(0)

    @pl.when(i == 0)
    def _():
        m_sc[...] = jnp.full_like(m_sc[...], -1e30)
        z_sc[...] = jnp.zeros_like(z_sc[...])

    px = pos_blk[:, 0:1]
    py = pos_blk[:, 1:2]
    xr = xcT[0:1, :]
    yr = xcT[1:2, :]
    xp = _bdot(pos_blk[...], xcT[0:2, :])
    d2 = (xr * xr + yr * yr + (px * px + py * py)) - 2.0 * xp
    logits = -_TAU * d2
    bm = jnp.max(logits, axis=0, keepdims=True)
    m_old = m_sc[...]
    m_new = jnp.maximum(m_old, bm)
    alpha = jnp.exp(m_old - m_new)
    z_sc[...] = z_sc[...] * alpha + jnp.sum(jnp.exp(logits - m_new), axis=0,
                                            keepdims=True)
    m_sc[...] = m_new

    @pl.when(i == _NB - 1)
    def _():
        m_out[...] = m_sc[...]
        z_out[...] = z_sc[...]


def _agg_body(xcT, pos_blk, emb, z_row, m_row, pose_blk, wrT, wsT, mb,
              out_blk, a_out, s_out):
    px = pos_blk[:, 0:1]
    py = pos_blk[:, 1:2]
    xr = xcT[0:1, :]
    yr = xcT[1:2, :]
    xp = _bdot(pos_blk[...], xcT[0:2, :])
    d2 = (xr * xr + yr * yr + (px * px + py * py)) - 2.0 * xp
    w = jnp.exp(-_TAU * d2 - m_row[...]) / z_row[...]
    nd = _bdot(w, emb[...]) + pose_blk[...]
    out_blk[...] = nd
    a_out[...] = _bdot(nd, wrT[...]) + mb[...]
    s_out[...] = _bdot(nd, wsT[...])


def _upd_core(nodes_blk, ib_blk, w1T, w2T, nbias, g2, b2, pose_blk):
    nd = nodes_blk[...]
    ib = ib_blk[0] + ib_blk[1]
    y = _bdot(nd, w1T[...]) + _bdot(ib, w2T[...]) + nbias[...]
    mu = jnp.sum(y, axis=1, keepdims=True) * (1.0 / 126.0)
    var = jnp.sum(y * y, axis=1, keepdims=True) * (1.0 / 126.0) - mu * mu
    rstd = lax.rsqrt(var + 1e-5)
    return (y - mu) * rstd * g2[...] + b2[...] + pose_blk[...]


def _upd_body(nodes_blk, ib_blk, w1T, w2T, nbias, g2, b2, pose_blk, out_blk):
    out_blk[...] = _upd_core(nodes_blk, ib_blk, w1T, w2T, nbias, g2, b2,
                             pose_blk)


def _upd_as_body(nodes_blk, ib_blk, w1T, w2T, nbias, g2, b2, pose_blk,
                 wrT, wsT, mb, out_blk, a_out, s_out):
    nd = _upd_core(nodes_blk, ib_blk, w1T, w2T, nbias, g2, b2, pose_blk)
    out_blk[...] = nd
    a_out[...] = _bdot(nd, wrT[...]) + mb[...]
    s_out[...] = _bdot(nd, wsT[...])


def _tgt_body(xt, posT_blk, lat_blk, xte, m_col, z_col,
              dw0, db0, dw1, db1, dw2, db2, out, acc_sc):
    i = pl.program_id(0)

    @pl.when(i == 0)
    def _():
        acc_sc[...] = jnp.zeros_like(acc_sc[...])

    xa = xt[:, 0:1]
    ya = xt[:, 1:2]
    pxr = posT_blk[0:1, :]
    pyr = posT_blk[1:2, :]
    xp = _bdot(xt[...], posT_blk[0:2, :])
    d2 = (xa * xa + ya * ya + (pxr * pxr + pyr * pyr)) - 2.0 * xp
    p = jnp.exp(-_TAU * d2 - m_col[...]) / z_col[...]
    acc_sc[...] = acc_sc[...] + _bdot(p, lat_blk[...])

    @pl.when(i == _NB - 1)
    def _():
        colmask = (lax.broadcasted_iota(jnp.int32, (1, 128), 1) >= 2
                   ).astype(_F32)
        zin = acc_sc[...] * colmask + xte[...]
        h = jnp.maximum(_bdot(zin, dw0[...]) + db0[...], 0.0)
        h = jnp.maximum(_bdot(h, dw1[...]) + db1[...], 0.0)
        out[...] = _bdot(h, dw2[...]) + db2[...]


def _rsqrt16(v):
    i = plsc.bitcast(v, jnp.int32)
    i = jnp.full((16,), 0x5F3759DF, jnp.int32) - jnp.right_shift(
        i, jnp.full((16,), 1, jnp.int32))
    y = plsc.bitcast(i, _F32)
    halfv = v * jnp.full((16,), 0.5, _F32)
    c15 = jnp.full((16,), 1.5, _F32)
    for _ in range(3):
        y = y * (c15 - halfv * y * y)
    return y


def _sc_body(a_t, s_t, ridx_h, sidx_h, g_h, b_h, out_h,
             ridx_v, sidx_v, a0, a1, s0, s1, gb_v, red0, red1, red2, red3,
             spm, ga0, ga1, gs0, gs1):
    core = lax.axis_index("c")
    sub = lax.axis_index("s")

    def _zrow(r, c):
        for k in range(8):
            a0[r, pl.ds(k * 16, 16)] = jnp.zeros((16,), _F32)
        return c
    lax.fori_loop(0, _CH, _zrow, 0)
    for t in range(640 // _CH):
        pltpu.sync_copy(a0, spm.at[pl.ds(sub * 640 + t * _CH, _CH)])
    plsc.subcore_barrier()

    pltpu.sync_copy(g_h, gb_v.at[0])
    pltpu.sync_copy(b_h, gb_v.at[1])
    gs = [gb_v[0, pl.ds(k * 16, 16)] for k in range(8)]
    bs = [gb_v[1, pl.ds(k * 16, 16)] for k in range(8)]

    pltpu.sync_copy(ridx_h.at[core, sub], ridx_v)
    pltpu.sync_copy(sidx_h.at[core, sub, pl.ds(0, _NCH // 2)], sidx_v)

    iota = lax.iota(jnp.int32, 16)
    perms = [jnp.bitwise_xor(iota, jnp.full((16,), k, jnp.int32))
             for k in (8, 4, 2, 1)]
    c_inv128 = jnp.full((16,), 1.0 / 128.0, _F32)
    c_eps = jnp.full((16,), 1e-5, _F32)

    def _allsum(v, slot):
        for p in perms:
            slot[...] = v
            v = v + plsc.load_gather(slot, [p])
        return v

    def _ln_rows(av, sv, ra, rb):
        def _row(r, cc):
            x = [av[r, pl.ds(k * 16, 16)] + sv[r, pl.ds(k * 16, 16)]
                 for k in range(8)]
            s = (((x[0] + x[1]) + (x[2] + x[3]))
                 + ((x[4] + x[5]) + (x[6] + x[7])))
            q = x[0] * x[0]
            for k in range(1, 8):
                q = q + x[k] * x[k]
            tot = _allsum(s, ra)
            qt = _allsum(q, rb)
            mu = tot * c_inv128
            var = qt * c_inv128 - mu * mu
            rv = _rsqrt16(var + c_eps)
            for k in range(8):
                av[r, pl.ds(k * 16, 16)] = (x[k] - mu) * (gs[k] * rv) + bs[k]
            return cc
        lax.fori_loop(0, _CH, _row, 0)

    def _srow(jj):
        return jnp.where(jj >= _NCH // 2, jj - _NCH // 2, jj)

    pltpu.async_copy(a_t.at[ridx_v.at[0]], a0, ga0)
    pltpu.async_copy(s_t.at[sidx_v.at[0]], s0, gs0)

    def _pair(t, c):
        j0 = t * 2
        j1 = j0 + 1
        pltpu.make_async_copy(a_t.at[ridx_v.at[0]], a0, ga0).wait()
        pltpu.make_async_copy(s_t.at[sidx_v.at[0]], s0, gs0).wait()
        pltpu.async_copy(a_t.at[ridx_v.at[j1]], a1, ga1)
        pltpu.async_copy(s_t.at[sidx_v.at[_srow(j1)]], s1, gs1)
        _ln_rows(a0, s0, red0, red1)
        pltpu.sync_copy(a0, spm.at[ridx_v.at[j0]], add=True)

        pltpu.make_async_copy(a_t.at[ridx_v.at[0]], a1, ga1).wait()
        pltpu.make_async_copy(s_t.at[sidx_v.at[0]], s1, gs1).wait()

        @pl.when(t == _NCH // 4 - 1)
        def _():
            pltpu.sync_copy(sidx_h.at[core, sub, pl.ds(_NCH // 2, _NCH // 2)],
                            sidx_v)

        @pl.when(t < _NCH // 2 - 1)
        def _():
            j2 = j0 + 2
            pltpu.async_copy(a_t.at[ridx_v.at[j2]], a0, ga0)
            pltpu.async_copy(s_t.at[sidx_v.at[_srow(j2)]], s0, gs0)
        _ln_rows(a1, s1, red2, red3)
        pltpu.sync_copy(a1, spm.at[ridx_v.at[j1]], add=True)
        return c
    lax.fori_loop(0, _NCH // 2, _pair, 0)

    plsc.subcore_barrier()
    pltpu.sync_copy(spm.at[pl.ds(sub * 640, 640)],
                    out_h.at[core, pl.ds(sub * 640, 640)])


def _edge_phase(a_t, s_t, ridx3, sidx3, g, b):
    mesh = plsc.VectorSubcoreMesh(core_axis_name="c", subcore_axis_name="s",
                                  num_cores=2, num_subcores=16)
    f = pl.kernel(
        _sc_body,
        out_type=jax.ShapeDtypeStruct((2, _NPAD, 128), _F32),
        mesh=mesh,
        scratch_types=[
            pltpu.VMEM((_NCH, _CH), jnp.int32),
            pltpu.VMEM((_NCH // 2, _CH), jnp.int32),
            pltpu.VMEM((_CH, 128), _F32),
            pltpu.VMEM((_CH, 128), _F32),
            pltpu.VMEM((_CH, 128), _F32),
            pltpu.VMEM((_CH, 128), _F32),
            pltpu.VMEM((2, 128), _F32),
            pltpu.VMEM((16,), _F32),
            pltpu.VMEM((16,), _F32),
            pltpu.VMEM((16,), _F32),
            pltpu.VMEM((16,), _F32),
            pltpu.VMEM_SHARED((_NPAD, 128), _F32),
            pltpu.SemaphoreType.DMA,
            pltpu.SemaphoreType.DMA,
            pltpu.SemaphoreType.DMA,
            pltpu.SemaphoreType.DMA,
        ],
        compiler_params=pltpu.CompilerParams(needs_layout_passes=False),
    )
    return f(a_t, s_t, ridx3, sidx3, g, b)


def _pad2d(a, rows, cols, row0=0, col0=0, fill=0.0):
    out = jnp.full((rows, cols), fill, a.dtype)
    return out.at[row0:row0 + a.shape[0], col0:col0 + a.shape[1]].set(a)


def kernel(xc, yc, xt, pos, senders, receivers,
           enc_W0, enc_b0, enc_W1, enc_b1, enc_W2, enc_b2,
           dec_W0, dec_b0, dec_W1, dec_b1, dec_W2, dec_b2,
           msg_W, msg_b, node_W, node_b,
           ln1_g, ln1_b, ln2_g, ln2_b):
    xc0, yc0, xt0 = xc[0], yc[0], xt[0]
    nc = xc0.shape[0]

    pos_far = jnp.concatenate(
        [pos, jnp.full((_NPAD - _N, 2), 1e3, _F32)], axis=0)
    pos_embed = _pad2d(pos, _NPAD, 128)
    xcT8 = _pad2d(xc0.T, 8, nc)
    posT8 = _pad2d(pos_far.T, 8, _NPAD)

    xtT8 = _pad2d(xt0.T, 8, nc)
    xcyc = _pad2d(jnp.concatenate([xc0, yc0], axis=1), nc, 8)
    ew0 = _pad2d(enc_W0.T, 8, 128)
    eb0 = _pad2d(enc_b0[None], 1, 128)
    ew1 = _pad2d(enc_W1.T, 128, 128)
    eb1 = _pad2d(enc_b1[None], 1, 128)
    ew2s = _pad2d(enc_W2.T, 128, 128, col0=2)
    eb2s = _pad2d(enc_b2[None], 1, 128, col0=2)

    mWT = msg_W.T
    wrT, wsT = mWT[:128], mWT[128:]
    mb = msg_b[None]

    nWT = node_W.T
    w1Ts = _pad2d(nWT[:128], 128, 128, col0=2)
    w2Ts = _pad2d(nWT[128:], 128, 128, col0=2)
    nbias = _pad2d(node_b[None], 1, 128, col0=2)
    g2s = _pad2d(ln2_g[None], 1, 128, col0=2)
    b2s = _pad2d(ln2_b[None], 1, 128, col0=2)

    dw0T = dec_W0.T
    dw0r = _pad2d(jnp.concatenate([dw0T[126:], dw0T[:126]], axis=0), 128, 128)
    db0 = _pad2d(dec_b0[None], 1, 128)
    dw1 = _pad2d(dec_W1.T, 128, 128)
    db1 = _pad2d(dec_b1[None], 1, 128)
    dw2 = _pad2d(dec_W2.T, 128, 128)
    db2 = _pad2d(dec_b2[None], 1, 128)
    xte = _pad2d(xt0, nc, 128)

    pad_e = _EPAD - _E
    send3 = jnp.concatenate(
        [senders, jnp.zeros((pad_e,), jnp.int32)]).reshape(2, 16, _NCH, _CH)
    recv3 = jnp.concatenate(
        [receivers, jnp.full((pad_e,), _N, jnp.int32)]).reshape(2, 16, _NCH, _CH)

    emb = pl.pallas_call(
        _enc_body,
        out_shape=jax.ShapeDtypeStruct((nc, 128), _F32),
    )(xcyc, ew0, eb0, ew1, eb1, ew2s, eb2s)

    full = lambda *_: (0, 0)

    def _stats(xT8):
        return pl.pallas_call(
            _stats_body,
            grid=(_NB,),
            in_specs=[pl.BlockSpec((8, nc), full),
                      pl.BlockSpec((_BN, 2), lambda i: (i, 0))],
            out_specs=[pl.BlockSpec((1, nc), full), pl.BlockSpec((1, nc), full)],
            out_shape=[jax.ShapeDtypeStruct((1, nc), _F32)] * 2,
            scratch_shapes=[pltpu.VMEM((1, nc), _F32),
                            pltpu.VMEM((1, nc), _F32)],
        )(xT8, pos_far)

    m_row, z_row = _stats(xcT8)
    m_rowt, z_rowt = _stats(xtT8)
    m_colt = m_rowt.reshape(nc, 1)
    z_colt = z_rowt.reshape(nc, 1)

    nodes, a_t, s_t = pl.pallas_call(
        _agg_body,
        grid=(_NB,),
        in_specs=[pl.BlockSpec((8, nc), full),
                  pl.BlockSpec((_BN, 2), lambda i: (i, 0)),
                  pl.BlockSpec((nc, 128), full),
                  pl.BlockSpec((1, nc), full),
                  pl.BlockSpec((1, nc), full),
                  pl.BlockSpec((_BN, 128), lambda i: (i, 0)),
                  pl.BlockSpec((128, 128), full),
                  pl.BlockSpec((128, 128), full),
                  pl.BlockSpec((1, 128), full)],
        out_specs=[pl.BlockSpec((_BN, 128), lambda i: (i, 0))] * 3,
        out_shape=[jax.ShapeDtypeStruct((_NPAD, 128), _F32)] * 3,
    )(xcT8, pos_far, emb, z_row, m_row, pos_embed, wrT, wsT, mb)

    for step in range(3):
        inbox2 = _edge_phase(a_t, s_t, recv3, send3, ln1_g, ln1_b)

        upd_in = [pl.BlockSpec((_BN, 128), lambda i: (i, 0)),
                  pl.BlockSpec((2, _BN, 128), lambda i: (0, i, 0)),
                  pl.BlockSpec((128, 128), full),
                  pl.BlockSpec((128, 128), full),
                  pl.BlockSpec((1, 128), full),
                  pl.BlockSpec((1, 128), full),
                  pl.BlockSpec((1, 128), full),
                  pl.BlockSpec((_BN, 128), lambda i: (i, 0))]
        if step < 2:
            nodes, a_t, s_t = pl.pallas_call(
                _upd_as_body,
                grid=(_NB,),
                in_specs=upd_in + [pl.BlockSpec((128, 128), full),
                                   pl.BlockSpec((128, 128), full),
                                   pl.BlockSpec((1, 128), full)],
                out_specs=[pl.BlockSpec((_BN, 128), lambda i: (i, 0))] * 3,
                out_shape=[jax.ShapeDtypeStruct((_NPAD, 128), _F32)] * 3,
            )(nodes, inbox2, w1Ts, w2Ts, nbias, g2s, b2s, pos_embed,
              wrT, wsT, mb)
        else:
            nodes = pl.pallas_call(
                _upd_body,
                grid=(_NB,),
                in_specs=upd_in,
                out_specs=pl.BlockSpec((_BN, 128), lambda i: (i, 0)),
                out_shape=jax.ShapeDtypeStruct((_NPAD, 128), _F32),
            )(nodes, inbox2, w1Ts, w2Ts, nbias, g2s, b2s, pos_embed)

    outp = pl.pallas_call(
        _tgt_body,
        grid=(_NB,),
        in_specs=[pl.BlockSpec((nc, 2), full),
                  pl.BlockSpec((8, _BN), lambda i: (0, i)),
                  pl.BlockSpec((_BN, 128), lambda i: (i, 0)),
                  pl.BlockSpec((nc, 128), full),
                  pl.BlockSpec((nc, 1), full),
                  pl.BlockSpec((nc, 1), full)]
                 + [pl.BlockSpec((128, 128), full),
                    pl.BlockSpec((1, 128), full)] * 3,
        out_specs=pl.BlockSpec((nc, 128), full),
        out_shape=jax.ShapeDtypeStruct((nc, 128), _F32),
        scratch_shapes=[pltpu.VMEM((nc, 128), _F32)],
    )(xt0, posT8, nodes, xte, m_colt, z_colt, dw0r, db0, dw1, db1, dw2, db2)

    return outp[:, :3][None]
